# baseline (speedup 1.0000x reference)
import functools

import jax
import jax.numpy as jnp
from jax.experimental import pallas as pl

N = 100000
H = 64
BN = 4000


def _dense1_body(agg1_ref, x_ref, w1rel_ref, b1_ref, w1root_ref, w2rel_ref,
                 h1_ref, t2_ref):
    agg1 = agg1_ref[...]
    x = x_ref[...]
    h1 = jnp.maximum(
        jnp.dot(agg1, w1rel_ref[...], preferred_element_type=jnp.float32)
        + b1_ref[...]
        + jnp.dot(x, w1root_ref[...], preferred_element_type=jnp.float32),
        0.0)
    h1_ref[...] = h1
    t2_ref[...] = jnp.dot(h1, w2rel_ref[...], preferred_element_type=jnp.float32)


def _dense2_body(agg2_ref, h1_ref, b2_ref, w2root_ref, w3rel_ref, b3_ref,
                 w3root_ref, t3_ref, r3_ref):
    h2 = jnp.maximum(
        agg2_ref[...] + b2_ref[...]
        + jnp.dot(h1_ref[...], w2root_ref[...], preferred_element_type=jnp.float32),
        0.0)
    t3_ref[...] = jnp.dot(h2, w3rel_ref[...], preferred_element_type=jnp.float32)
    r3_ref[...] = b3_ref[...] + jnp.dot(h2, w3root_ref[...],
                                        preferred_element_type=jnp.float32)


def _row_block(width):
    return pl.BlockSpec((BN, width), lambda i: (i, 0))


def _full(shape):
    return pl.BlockSpec(shape, lambda i: tuple(0 for _ in shape))


def kernel(x, edge_index, edge_weight, batch, W1_rel, b1, W1_root, W2_rel, b2,
           W2_root, W3_rel, b3, W3_root):
    del batch
    src = edge_index[0]
    dst = edge_index[1]
    w = edge_weight[:, None]

    agg1 = jax.ops.segment_sum(x[src] * w, dst, num_segments=N)

    dense1 = pl.pallas_call(
        _dense1_body,
        grid=(N // BN,),
        in_specs=[_row_block(2), _row_block(2), _full((2, H)), _full((1, H)),
                  _full((2, H)), _full((H, H))],
        out_specs=[_row_block(H), _row_block(H)],
        out_shape=[jax.ShapeDtypeStruct((N, H), jnp.float32),
                   jax.ShapeDtypeStruct((N, H), jnp.float32)],
    )
    h1, t2 = dense1(agg1, x, W1_rel, b1.reshape(1, H), W1_root, W2_rel)

    agg2 = jax.ops.segment_sum(t2[src] * w, dst, num_segments=N)

    dense2 = pl.pallas_call(
        _dense2_body,
        grid=(N // BN,),
        in_specs=[_row_block(H), _row_block(H), _full((1, H)), _full((H, H)),
                  _full((H, 3)), _full((1, 3)), _full((H, 3))],
        out_specs=[_row_block(3), _row_block(3)],
        out_shape=[jax.ShapeDtypeStruct((N, 3), jnp.float32),
                   jax.ShapeDtypeStruct((N, 3), jnp.float32)],
    )
    t3, r3 = dense2(agg2, h1, b2.reshape(1, H), W2_root, W3_rel,
                    b3.reshape(1, 3), W3_root)

    agg3 = jax.ops.segment_sum(t3[src] * w, dst, num_segments=N)
    return agg3 + r3


# speedup vs baseline: 9.7784x; 9.7784x over previous
import functools

import jax
import jax.numpy as jnp
from jax import lax
from jax.experimental import pallas as pl
from jax.experimental.pallas import tpu as pltpu
from jax.experimental.pallas import tpu_sc as plsc

N = 100000
NP = 100096
E = 1600000
H = 64
F = 16
NC = 2
NS = 16
SCAT = 125
RBLK = 1000
BN = 4000

ROWS_PER_TILE = NP // NS


def _sc_agg_body(n_items, blocks_per_tile, blk, edge_base_fn, chunk_of_item,
                 out_base_fn, table, src, dst2, w, out, acc, rows, src_v,
                 dst_v, w_v, sem):
    c = lax.axis_index("c")
    s = lax.axis_index("s")

    zero16 = jnp.zeros((F,), jnp.float32)

    def _tile_slice_sizes():
        sizes = []
        rem = ROWS_PER_TILE
        while rem > 0:
            sizes.append(min(RBLK, rem))
            rem -= sizes[-1]
        return sizes

    for j in range(n_items):
        @pl.loop(0, RBLK, unroll=8)
        def _zero_rows(i):
            rows[i, :] = zero16

        off = 0
        for sz in _tile_slice_sizes():
            dst_off = pl.multiple_of(s * ROWS_PER_TILE + off, 8)
            pltpu.sync_copy(rows.at[pl.ds(0, sz)], acc.at[pl.ds(dst_off, sz)])
            off += sz
        plsc.subcore_barrier()

        ebase = edge_base_fn(c, s, j)
        cbase = chunk_of_item(c, j)

        @pl.loop(0, blocks_per_tile)
        def _block(b):
            base = pl.multiple_of(ebase + b * blk, 8)
            rbase = pl.multiple_of((ebase + b * blk) // SCAT, 8)
            pltpu.sync_copy(src.at[pl.ds(base, blk)], src_v)
            pltpu.sync_copy(w.at[pl.ds(base, blk)], w_v.at[pl.ds(0, blk)])
            pltpu.sync_copy(dst2.at[pl.ds(rbase, blk // SCAT)], dst_v)

            if cbase is not None:
                @pl.loop(0, blk // F, unroll=8)
                def _adj(i):
                    src_v[pl.ds(i * F, F)] = src_v[pl.ds(i * F, F)] + cbase

            for k0 in range(0, blk, RBLK):
                pltpu.async_copy(table.at[src_v.at[pl.ds(k0, RBLK)]], rows,
                                 sem).wait()

                ngrp = RBLK // F

                @pl.loop(0, ngrp)
                def _scale(g):
                    wv = w_v[pl.ds(k0 + g * F, F)]
                    base16 = g * F
                    for l in range(F):
                        rows[base16 + l, :] = rows[base16 + l, :] * wv[l]

                if RBLK % F:
                    wv = w_v[pl.ds(k0 + ngrp * F, F)]
                    for l in range(RBLK % F):
                        rows[ngrp * F + l, :] = rows[ngrp * F + l, :] * wv[l]

                for k in range(RBLK // SCAT):
                    pltpu.sync_copy(rows.at[pl.ds(k * SCAT, SCAT)],
                                    acc.at[dst_v.at[k0 // SCAT + k]],
                                    add=True)

        plsc.subcore_barrier()

        obase = out_base_fn(c, j)
        off = 0
        for sz in _tile_slice_sizes():
            row_off = pl.multiple_of(s * ROWS_PER_TILE + off, 8)
            pltpu.sync_copy(acc.at[pl.ds(row_off, sz)],
                            out.at[pl.ds(obase + row_off, sz)])
            off += sz
        plsc.subcore_barrier()


def _make_sc_agg(n_chunks, blk):
    if n_chunks == 1:
        n_items = 1
        per_tile = E // (NC * NS)

        def edge_base(c, s, j):
            del j
            return c * (E // NC) + s * per_tile

        def chunk_of(c, j):
            return None

        def out_base(c, j):
            del j
            return c * NP
    else:
        n_items = n_chunks // NC
        per_tile = E // NS

        def edge_base(c, s, j):
            del c, j
            return s * per_tile

        def chunk_of(c, j):
            return (c * n_items + j) * NP

        def out_base(c, j):
            return (c * n_items + j) * NP

    body = functools.partial(_sc_agg_body, n_items, per_tile // blk, blk,
                             edge_base, chunk_of, out_base)
    n_out_chunks = 2 if n_chunks == 1 else n_chunks
    return pl.kernel(
        body,
        out_type=jax.ShapeDtypeStruct((n_out_chunks * NP, F), jnp.float32),
        mesh=plsc.VectorSubcoreMesh(core_axis_name="c", subcore_axis_name="s"),
        compiler_params=pltpu.CompilerParams(use_tc_tiling_on_sc=False),
        scratch_types=[
            pltpu.VMEM_SHARED((NP, F), jnp.float32),
            pltpu.VMEM((RBLK, F), jnp.float32),
            pltpu.VMEM((blk,), jnp.int32),
            pltpu.VMEM((blk // SCAT, SCAT), jnp.int32),
            pltpu.VMEM((blk + F,), jnp.float32),
            pltpu.SemaphoreType.DMA,
        ],
    )


_agg_narrow = _make_sc_agg(1, 2000)
_agg_wide = _make_sc_agg(4, 4000)


def _dense1_body(agg1_ref, x_ref, w1rel_ref, b1_ref, w1root_ref, w2rel_ref,
                 h1_ref, t2_ref):
    agg1 = agg1_ref[0] + agg1_ref[1]
    h1 = jnp.maximum(
        jnp.dot(agg1, w1rel_ref[...], preferred_element_type=jnp.float32)
        + b1_ref[...]
        + jnp.dot(x_ref[...], w1root_ref[...],
                  preferred_element_type=jnp.float32),
        0.0)
    h1_ref[...] = h1
    t2 = jnp.dot(h1, w2rel_ref[...], preferred_element_type=jnp.float32)
    for ch in range(4):
        t2_ref[ch] = t2[:, ch * F:(ch + 1) * F]


def _dense2_body(agg2_ref, h1_ref, b2_ref, w2root_ref, w3rel_ref, b3_ref,
                 w3root_ref, t3_ref, r3_ref):
    agg2 = jnp.concatenate([agg2_ref[ch] for ch in range(4)], axis=-1)
    h2 = jnp.maximum(
        agg2 + b2_ref[...]
        + jnp.dot(h1_ref[...], w2root_ref[...],
                  preferred_element_type=jnp.float32),
        0.0)
    t3_ref[...] = jnp.dot(h2, w3rel_ref[...], preferred_element_type=jnp.float32)
    r3_ref[...] = b3_ref[...] + jnp.dot(h2, w3root_ref[...],
                                        preferred_element_type=jnp.float32)


def _dense3_body(agg3_ref, r3_ref, out_ref):
    out_ref[...] = agg3_ref[0] + agg3_ref[1] + r3_ref[...]


def _row_block(width):
    return pl.BlockSpec((BN, width), lambda i: (i, 0))


def _chunk_block():
    return pl.BlockSpec((4, BN, F), lambda i: (0, i, 0))


def _pair_block():
    return pl.BlockSpec((2, BN, F), lambda i: (0, i, 0))


def _full(shape):
    return pl.BlockSpec(shape, lambda i: tuple(0 for _ in shape))


def kernel(x, edge_index, edge_weight, batch, W1_rel, b1, W1_root, W2_rel, b2,
           W2_root, W3_rel, b3, W3_root):
    del batch
    src = edge_index[0].astype(jnp.int32)
    dst2 = edge_index[1].astype(jnp.int32).reshape(E // SCAT, SCAT)
    w = edge_weight

    x16 = jnp.pad(x, ((0, NP - N), (0, F - 2)))
    W1_rel16 = jnp.pad(W1_rel, ((0, F - 2), (0, 0)))
    W3_rel16 = jnp.pad(W3_rel, ((0, 0), (0, F - 3)))
    W3_root16 = jnp.pad(W3_root, ((0, 0), (0, F - 3)))
    b3_16 = jnp.pad(b3, (0, F - 3)).reshape(1, F)

    agg1 = _agg_narrow(x16, src, dst2, w).reshape(2, NP, F)

    dense1 = pl.pallas_call(
        _dense1_body,
        grid=(N // BN,),
        in_specs=[_pair_block(), _row_block(2), _full((F, H)), _full((1, H)),
                  _full((2, H)), _full((H, H))],
        out_specs=[_row_block(H), _chunk_block()],
        out_shape=[jax.ShapeDtypeStruct((N, H), jnp.float32),
                   jax.ShapeDtypeStruct((4, NP, F), jnp.float32)],
    )
    h1, t2c = dense1(agg1, x, W1_rel16, b1.reshape(1, H), W1_root, W2_rel)

    agg2 = _agg_wide(t2c.reshape(4 * NP, F), src, dst2, w).reshape(4, NP, F)

    dense2 = pl.pallas_call(
        _dense2_body,
        grid=(N // BN,),
        in_specs=[_chunk_block(), _row_block(H), _full((1, H)), _full((H, H)),
                  _full((H, F)), _full((1, F)), _full((H, F))],
        out_specs=[_row_block(F), _row_block(F)],
        out_shape=[jax.ShapeDtypeStruct((NP, F), jnp.float32),
                   jax.ShapeDtypeStruct((N, F), jnp.float32)],
    )
    t3, r3 = dense2(agg2, h1, b2.reshape(1, H), W2_root, W3_rel16, b3_16,
                    W3_root16)

    agg3 = _agg_narrow(t3, src, dst2, w).reshape(2, NP, F)

    dense3 = pl.pallas_call(
        _dense3_body,
        grid=(N // BN,),
        in_specs=[_pair_block(), _row_block(F)],
        out_specs=_row_block(F),
        out_shape=jax.ShapeDtypeStruct((N, F), jnp.float32),
    )
    out16 = dense3(agg3, r3)
    return out16[:, :3]


# speedup vs baseline: 10.5748x; 1.0814x over previous
import functools

import jax
import jax.numpy as jnp
from jax import lax
from jax.experimental import pallas as pl
from jax.experimental.pallas import tpu as pltpu
from jax.experimental.pallas import tpu_sc as plsc

N = 100000
NP = 100096
E = 1600000
H = 64
F = 16
NC = 2
NS = 16
SCAT = 125
RBLK = 1000
BN = 4000

ROWS_PER_TILE = NP // NS


def _sc_agg_body(n_items, blocks_per_tile, blk, edge_base_fn, chunk_of_item,
                 out_base_fn, table, src, dst2, w, out, acc, rows,
                 src_v0, src_v1, dst_v0, dst_v1, w_v0, w_v1,
                 insem0, insem1, gsem):
    src_vs = (src_v0, src_v1)
    dst_vs = (dst_v0, dst_v1)
    w_vs = (w_v0, w_v1)
    insems = (insem0, insem1)
    c = lax.axis_index("c")
    s = lax.axis_index("s")

    zero16 = jnp.zeros((F,), jnp.float32)

    def _in_copies(eb, par):
        base = pl.multiple_of(eb, 8)
        rbase = pl.multiple_of(eb // SCAT, 8)
        return (
            pltpu.make_async_copy(src.at[pl.ds(base, blk)], src_vs[par],
                                  insems[par]),
            pltpu.make_async_copy(w.at[pl.ds(base, blk)],
                                  w_vs[par].at[pl.ds(0, blk)], insems[par]),
            pltpu.make_async_copy(dst2.at[pl.ds(rbase, blk // SCAT)],
                                  dst_vs[par], insems[par]),
        )

    def _tile_slice_sizes():
        sizes = []
        rem = ROWS_PER_TILE
        while rem > 0:
            sizes.append(min(RBLK, rem))
            rem -= sizes[-1]
        return sizes

    for j in range(n_items):
        @pl.loop(0, RBLK, unroll=8)
        def _zero_rows(i):
            rows[i, :] = zero16

        off = 0
        for sz in _tile_slice_sizes():
            dst_off = pl.multiple_of(s * ROWS_PER_TILE + off, 8)
            pltpu.sync_copy(rows.at[pl.ds(0, sz)], acc.at[pl.ds(dst_off, sz)])
            off += sz
        plsc.subcore_barrier()

        ebase = edge_base_fn(c, s, j)
        cbase = chunk_of_item(c, j)

        for par in (0, 1):
            for d in _in_copies(ebase + par * blk, par):
                d.start()

        @pl.loop(0, blocks_per_tile // 2)
        def _block2(i):
            for par in (0, 1):
                b = i * 2 + par
                eb = ebase + b * blk
                for d in _in_copies(eb, par):
                    d.wait()

                src_v, dst_v, w_v = src_vs[par], dst_vs[par], w_vs[par]
                if cbase is not None:
                    @pl.loop(0, blk // F, unroll=8)
                    def _adj(i2):
                        src_v[pl.ds(i2 * F, F)] = (
                            src_v[pl.ds(i2 * F, F)] + cbase)

                pltpu.async_copy(table.at[src_v], rows, gsem).wait()

                ngrp = blk // F

                @pl.loop(0, ngrp, unroll=2)
                def _scale(g):
                    wv = w_v[pl.ds(g * F, F)]
                    base16 = g * F
                    for l in range(F):
                        rows[base16 + l, :] = rows[base16 + l, :] * wv[l]

                if blk % F:
                    wv = w_v[pl.ds(ngrp * F, F)]
                    for l in range(blk % F):
                        rows[ngrp * F + l, :] = rows[ngrp * F + l, :] * wv[l]

                for t in range(blk // SCAT):
                    pltpu.sync_copy(rows.at[pl.ds(t * SCAT, SCAT)],
                                    acc.at[dst_v.at[t]], add=True)

                @pl.when(b + 2 < blocks_per_tile)
                def _prefetch():
                    for d in _in_copies(eb + 2 * blk, par):
                        d.start()

        plsc.subcore_barrier()

        obase = out_base_fn(c, j)
        off = 0
        for sz in _tile_slice_sizes():
            row_off = pl.multiple_of(s * ROWS_PER_TILE + off, 8)
            pltpu.sync_copy(acc.at[pl.ds(row_off, sz)],
                            out.at[pl.ds(obase + row_off, sz)])
            off += sz
        plsc.subcore_barrier()


def _make_sc_agg(n_chunks, blk):
    if n_chunks == 1:
        n_items = 1
        per_tile = E // (NC * NS)

        def edge_base(c, s, j):
            del j
            return c * (E // NC) + s * per_tile

        def chunk_of(c, j):
            return None

        def out_base(c, j):
            del j
            return c * NP
    else:
        n_items = n_chunks // NC
        per_tile = E // NS

        def edge_base(c, s, j):
            del c, j
            return s * per_tile

        def chunk_of(c, j):
            return (c * n_items + j) * NP

        def out_base(c, j):
            return (c * n_items + j) * NP

    body = functools.partial(_sc_agg_body, n_items, per_tile // blk, blk,
                             edge_base, chunk_of, out_base)
    n_out_chunks = 2 if n_chunks == 1 else n_chunks
    return pl.kernel(
        body,
        out_type=jax.ShapeDtypeStruct((n_out_chunks * NP, F), jnp.float32),
        mesh=plsc.VectorSubcoreMesh(core_axis_name="c", subcore_axis_name="s"),
        compiler_params=pltpu.CompilerParams(use_tc_tiling_on_sc=False),
        scratch_types=[
            pltpu.VMEM_SHARED((NP, F), jnp.float32),
            pltpu.VMEM((RBLK, F), jnp.float32),
            pltpu.VMEM((blk,), jnp.int32),
            pltpu.VMEM((blk,), jnp.int32),
            pltpu.VMEM((blk // SCAT, SCAT), jnp.int32),
            pltpu.VMEM((blk // SCAT, SCAT), jnp.int32),
            pltpu.VMEM((blk + F,), jnp.float32),
            pltpu.VMEM((blk + F,), jnp.float32),
            pltpu.SemaphoreType.DMA,
            pltpu.SemaphoreType.DMA,
            pltpu.SemaphoreType.DMA,
        ],
    )


_agg_narrow = _make_sc_agg(1, 1000)
_agg_wide = _make_sc_agg(4, 1000)


def _dense1_body(agg1_ref, x_ref, w1rel_ref, b1_ref, w1root_ref, w2rel_ref,
                 h1_ref, t2_ref):
    agg1 = agg1_ref[0] + agg1_ref[1]
    h1 = jnp.maximum(
        jnp.dot(agg1, w1rel_ref[...], preferred_element_type=jnp.float32)
        + b1_ref[...]
        + jnp.dot(x_ref[...], w1root_ref[...],
                  preferred_element_type=jnp.float32),
        0.0)
    h1_ref[...] = h1
    t2 = jnp.dot(h1, w2rel_ref[...], preferred_element_type=jnp.float32)
    for ch in range(4):
        t2_ref[ch] = t2[:, ch * F:(ch + 1) * F]


def _dense2_body(agg2_ref, h1_ref, b2_ref, w2root_ref, w3rel_ref, b3_ref,
                 w3root_ref, t3_ref, r3_ref):
    agg2 = jnp.concatenate([agg2_ref[ch] for ch in range(4)], axis=-1)
    h2 = jnp.maximum(
        agg2 + b2_ref[...]
        + jnp.dot(h1_ref[...], w2root_ref[...],
                  preferred_element_type=jnp.float32),
        0.0)
    t3_ref[...] = jnp.dot(h2, w3rel_ref[...], preferred_element_type=jnp.float32)
    r3_ref[...] = b3_ref[...] + jnp.dot(h2, w3root_ref[...],
                                        preferred_element_type=jnp.float32)


def _dense3_body(agg3_ref, r3_ref, out_ref):
    out_ref[...] = agg3_ref[0] + agg3_ref[1] + r3_ref[...]


def _row_block(width):
    return pl.BlockSpec((BN, width), lambda i: (i, 0))


def _chunk_block():
    return pl.BlockSpec((4, BN, F), lambda i: (0, i, 0))


def _pair_block():
    return pl.BlockSpec((2, BN, F), lambda i: (0, i, 0))


def _full(shape):
    return pl.BlockSpec(shape, lambda i: tuple(0 for _ in shape))


def kernel(x, edge_index, edge_weight, batch, W1_rel, b1, W1_root, W2_rel, b2,
           W2_root, W3_rel, b3, W3_root):
    del batch
    src = edge_index[0].astype(jnp.int32)
    dst2 = edge_index[1].astype(jnp.int32).reshape(E // SCAT, SCAT)
    w = edge_weight

    x16 = jnp.pad(x, ((0, NP - N), (0, F - 2)))
    W1_rel16 = jnp.pad(W1_rel, ((0, F - 2), (0, 0)))
    W3_rel16 = jnp.pad(W3_rel, ((0, 0), (0, F - 3)))
    W3_root16 = jnp.pad(W3_root, ((0, 0), (0, F - 3)))
    b3_16 = jnp.pad(b3, (0, F - 3)).reshape(1, F)

    agg1 = _agg_narrow(x16, src, dst2, w).reshape(2, NP, F)

    dense1 = pl.pallas_call(
        _dense1_body,
        grid=(N // BN,),
        in_specs=[_pair_block(), _row_block(2), _full((F, H)), _full((1, H)),
                  _full((2, H)), _full((H, H))],
        out_specs=[_row_block(H), _chunk_block()],
        out_shape=[jax.ShapeDtypeStruct((N, H), jnp.float32),
                   jax.ShapeDtypeStruct((4, NP, F), jnp.float32)],
    )
    h1, t2c = dense1(agg1, x, W1_rel16, b1.reshape(1, H), W1_root, W2_rel)

    agg2 = _agg_wide(t2c.reshape(4 * NP, F), src, dst2, w).reshape(4, NP, F)

    dense2 = pl.pallas_call(
        _dense2_body,
        grid=(N // BN,),
        in_specs=[_chunk_block(), _row_block(H), _full((1, H)), _full((H, H)),
                  _full((H, F)), _full((1, F)), _full((H, F))],
        out_specs=[_row_block(F), _row_block(F)],
        out_shape=[jax.ShapeDtypeStruct((NP, F), jnp.float32),
                   jax.ShapeDtypeStruct((N, F), jnp.float32)],
    )
    t3, r3 = dense2(agg2, h1, b2.reshape(1, H), W2_root, W3_rel16, b3_16,
                    W3_root16)

    agg3 = _agg_narrow(t3, src, dst2, w).reshape(2, NP, F)

    dense3 = pl.pallas_call(
        _dense3_body,
        grid=(N // BN,),
        in_specs=[_pair_block(), _row_block(F)],
        out_specs=_row_block(F),
        out_shape=jax.ShapeDtypeStruct((N, F), jnp.float32),
    )
    out16 = dense3(agg3, r3)
    return out16[:, :3]


# speedup vs baseline: 10.5757x; 1.0001x over previous
import functools

import jax
import jax.numpy as jnp
from jax import lax
from jax.experimental import pallas as pl
from jax.experimental.pallas import tpu as pltpu
from jax.experimental.pallas import tpu_sc as plsc

N = 100000
NP = 100096
E = 1600000
H = 64
F = 16
NC = 2
NS = 16
SCAT = 125
RBLK = 1000
BN = 4000

ROWS_PER_TILE = NP // NS


def _sc_agg_body(n_items, blocks_per_tile, blk, edge_base_fn, chunk_of_item,
                 out_base_fn, table, src, dst2, w, out, acc, rows,
                 src_v0, src_v1, dst_v0, dst_v1, w_v0, w_v1,
                 insem0, insem1, gsem):
    src_vs = (src_v0, src_v1)
    dst_vs = (dst_v0, dst_v1)
    w_vs = (w_v0, w_v1)
    insems = (insem0, insem1)
    c = lax.axis_index("c")
    s = lax.axis_index("s")

    zero16 = jnp.zeros((F,), jnp.float32)

    def _in_copies(eb, par):
        base = pl.multiple_of(eb, 8)
        rbase = pl.multiple_of(eb // SCAT, 8)
        return (
            pltpu.make_async_copy(src.at[pl.ds(base, blk)],
                                  src_vs[par].at[pl.ds(0, blk)], insems[par]),
            pltpu.make_async_copy(w.at[pl.ds(base, blk)],
                                  w_vs[par].at[pl.ds(0, blk)], insems[par]),
            pltpu.make_async_copy(dst2.at[pl.ds(rbase, blk // SCAT)],
                                  dst_vs[par], insems[par]),
        )

    def _tile_slice_sizes():
        sizes = []
        rem = ROWS_PER_TILE
        while rem > 0:
            sizes.append(min(RBLK, rem))
            rem -= sizes[-1]
        return sizes

    for j in range(n_items):
        @pl.loop(0, RBLK, unroll=8)
        def _zero_rows(i):
            rows[i, :] = zero16

        off = 0
        for sz in _tile_slice_sizes():
            dst_off = pl.multiple_of(s * ROWS_PER_TILE + off, 8)
            pltpu.sync_copy(rows.at[pl.ds(0, sz)], acc.at[pl.ds(dst_off, sz)])
            off += sz
        plsc.subcore_barrier()

        ebase = edge_base_fn(c, s, j)
        cbase = chunk_of_item(c, j)

        for par in (0, 1):
            for d in _in_copies(ebase + par * blk, par):
                d.start()

        @pl.loop(0, blocks_per_tile // 2)
        def _block2(i):
            for par in (0, 1):
                b = i * 2 + par
                eb = ebase + b * blk
                for d in _in_copies(eb, par):
                    d.wait()

                src_v, dst_v, w_v = src_vs[par], dst_vs[par], w_vs[par]
                if cbase is not None:
                    @pl.loop(0, blk // F, unroll=8)
                    def _adj(i2):
                        src_v[pl.ds(i2 * F, F)] = (
                            src_v[pl.ds(i2 * F, F)] + cbase)

                    if blk % F:
                        tb = (blk // F) * F
                        src_v[pl.ds(tb, F)] = src_v[pl.ds(tb, F)] + cbase

                pltpu.async_copy(table.at[src_v.at[pl.ds(0, blk)]], rows,
                                 gsem).wait()

                ngrp = blk // F

                @pl.loop(0, ngrp, unroll=2)
                def _scale(g):
                    wv = w_v[pl.ds(g * F, F)]
                    base16 = g * F
                    for l in range(F):
                        rows[base16 + l, :] = rows[base16 + l, :] * wv[l]

                if blk % F:
                    wv = w_v[pl.ds(ngrp * F, F)]
                    for l in range(blk % F):
                        rows[ngrp * F + l, :] = rows[ngrp * F + l, :] * wv[l]

                for t in range(blk // SCAT):
                    pltpu.sync_copy(rows.at[pl.ds(t * SCAT, SCAT)],
                                    acc.at[dst_v.at[t]], add=True)

                @pl.when(b + 2 < blocks_per_tile)
                def _prefetch():
                    for d in _in_copies(eb + 2 * blk, par):
                        d.start()

        plsc.subcore_barrier()

        obase = out_base_fn(c, j)
        off = 0
        for sz in _tile_slice_sizes():
            row_off = pl.multiple_of(s * ROWS_PER_TILE + off, 8)
            pltpu.sync_copy(acc.at[pl.ds(row_off, sz)],
                            out.at[pl.ds(obase + row_off, sz)])
            off += sz
        plsc.subcore_barrier()


def _make_sc_agg(n_chunks, blk):
    if n_chunks == 1:
        n_items = 1
        per_tile = E // (NC * NS)

        def edge_base(c, s, j):
            del j
            return c * (E // NC) + s * per_tile

        def chunk_of(c, j):
            return None

        def out_base(c, j):
            del j
            return c * NP
    else:
        n_items = n_chunks // NC
        per_tile = E // NS

        def edge_base(c, s, j):
            del c, j
            return s * per_tile

        def chunk_of(c, j):
            return (c * n_items + j) * NP

        def out_base(c, j):
            return (c * n_items + j) * NP

    body = functools.partial(_sc_agg_body, n_items, per_tile // blk, blk,
                             edge_base, chunk_of, out_base)
    n_out_chunks = 2 if n_chunks == 1 else n_chunks
    return pl.kernel(
        body,
        out_type=jax.ShapeDtypeStruct((n_out_chunks * NP, F), jnp.float32),
        mesh=plsc.VectorSubcoreMesh(core_axis_name="c", subcore_axis_name="s"),
        compiler_params=pltpu.CompilerParams(use_tc_tiling_on_sc=False),
        scratch_types=[
            pltpu.VMEM_SHARED((NP, F), jnp.float32),
            pltpu.VMEM((RBLK, F), jnp.float32),
            pltpu.VMEM((blk + F,), jnp.int32),
            pltpu.VMEM((blk + F,), jnp.int32),
            pltpu.VMEM((blk // SCAT, SCAT), jnp.int32),
            pltpu.VMEM((blk // SCAT, SCAT), jnp.int32),
            pltpu.VMEM((blk + F,), jnp.float32),
            pltpu.VMEM((blk + F,), jnp.float32),
            pltpu.SemaphoreType.DMA,
            pltpu.SemaphoreType.DMA,
            pltpu.SemaphoreType.DMA,
        ],
    )


_agg_narrow = _make_sc_agg(1, 1000)
_agg_wide = _make_sc_agg(4, 1000)


def _dense1_body(agg1_ref, x_ref, w1rel_ref, b1_ref, w1root_ref, w2rel_ref,
                 h1_ref, t2_ref):
    agg1 = agg1_ref[0] + agg1_ref[1]
    h1 = jnp.maximum(
        jnp.dot(agg1, w1rel_ref[...], preferred_element_type=jnp.float32)
        + b1_ref[...]
        + jnp.dot(x_ref[...], w1root_ref[...],
                  preferred_element_type=jnp.float32),
        0.0)
    h1_ref[...] = h1
    t2 = jnp.dot(h1, w2rel_ref[...], preferred_element_type=jnp.float32)
    for ch in range(4):
        t2_ref[ch] = t2[:, ch * F:(ch + 1) * F]


def _dense2_body(agg2_ref, h1_ref, b2_ref, w2root_ref, w3rel_ref, b3_ref,
                 w3root_ref, t3_ref, r3_ref):
    agg2 = jnp.concatenate([agg2_ref[ch] for ch in range(4)], axis=-1)
    h2 = jnp.maximum(
        agg2 + b2_ref[...]
        + jnp.dot(h1_ref[...], w2root_ref[...],
                  preferred_element_type=jnp.float32),
        0.0)
    t3_ref[...] = jnp.dot(h2, w3rel_ref[...], preferred_element_type=jnp.float32)
    r3_ref[...] = b3_ref[...] + jnp.dot(h2, w3root_ref[...],
                                        preferred_element_type=jnp.float32)


def _dense3_body(agg3_ref, r3_ref, out_ref):
    out_ref[...] = agg3_ref[0] + agg3_ref[1] + r3_ref[...]


def _row_block(width):
    return pl.BlockSpec((BN, width), lambda i: (i, 0))


def _chunk_block():
    return pl.BlockSpec((4, BN, F), lambda i: (0, i, 0))


def _pair_block():
    return pl.BlockSpec((2, BN, F), lambda i: (0, i, 0))


def _full(shape):
    return pl.BlockSpec(shape, lambda i: tuple(0 for _ in shape))


def kernel(x, edge_index, edge_weight, batch, W1_rel, b1, W1_root, W2_rel, b2,
           W2_root, W3_rel, b3, W3_root):
    del batch
    src = edge_index[0].astype(jnp.int32)
    dst2 = edge_index[1].astype(jnp.int32).reshape(E // SCAT, SCAT)
    w = edge_weight

    x16 = jnp.pad(x, ((0, NP - N), (0, F - 2)))
    W1_rel16 = jnp.pad(W1_rel, ((0, F - 2), (0, 0)))
    W3_rel16 = jnp.pad(W3_rel, ((0, 0), (0, F - 3)))
    W3_root16 = jnp.pad(W3_root, ((0, 0), (0, F - 3)))
    b3_16 = jnp.pad(b3, (0, F - 3)).reshape(1, F)

    agg1 = _agg_narrow(x16, src, dst2, w).reshape(2, NP, F)

    dense1 = pl.pallas_call(
        _dense1_body,
        grid=(N // BN,),
        in_specs=[_pair_block(), _row_block(2), _full((F, H)), _full((1, H)),
                  _full((2, H)), _full((H, H))],
        out_specs=[_row_block(H), _chunk_block()],
        out_shape=[jax.ShapeDtypeStruct((N, H), jnp.float32),
                   jax.ShapeDtypeStruct((4, NP, F), jnp.float32)],
    )
    h1, t2c = dense1(agg1, x, W1_rel16, b1.reshape(1, H), W1_root, W2_rel)

    agg2 = _agg_wide(t2c.reshape(4 * NP, F), src, dst2, w).reshape(4, NP, F)

    dense2 = pl.pallas_call(
        _dense2_body,
        grid=(N // BN,),
        in_specs=[_chunk_block(), _row_block(H), _full((1, H)), _full((H, H)),
                  _full((H, F)), _full((1, F)), _full((H, F))],
        out_specs=[_row_block(F), _row_block(F)],
        out_shape=[jax.ShapeDtypeStruct((NP, F), jnp.float32),
                   jax.ShapeDtypeStruct((N, F), jnp.float32)],
    )
    t3, r3 = dense2(agg2, h1, b2.reshape(1, H), W2_root, W3_rel16, b3_16,
                    W3_root16)

    agg3 = _agg_narrow(t3, src, dst2, w).reshape(2, NP, F)

    dense3 = pl.pallas_call(
        _dense3_body,
        grid=(N // BN,),
        in_specs=[_pair_block(), _row_block(F)],
        out_specs=_row_block(F),
        out_shape=jax.ShapeDtypeStruct((N, F), jnp.float32),
    )
    out16 = dense3(agg3, r3)
    return out16[:, :3]
